# Initial kernel scaffold; baseline (speedup 1.0000x reference)
import jax
import jax.numpy as jnp
from jax.experimental import pallas as pl


def kernel(pos_u, pos_v, neg_u, neg_v, u_emb, v_emb):
    raise NotImplementedError("write your pallas kernel here")


# speedup vs baseline: 8.1199x; 8.1199x over previous
import functools

import jax
import jax.numpy as jnp
from jax import lax
from jax.experimental import pallas as pl
from jax.experimental.pallas import tpu as pltpu
from jax.experimental.pallas import tpu_sc as plsc

VOCAB = 100000
DIM = 128
B = 16384
BN = 81920
CTX = 20
NTOT = B + BN

NC = 2
NS = 16
NW = NC * NS
EW = NTOT // NW
CK = 16
NCH = EW // CK
L = 16


def _sc_scores_body(uidx_hbm, vidx_hbm, u_hbm, v_hbm, out_hbm,
                    uidx_v, vidx_v, urows_v, vrows_v, psum_v, osc_v, sem):
    wid = lax.axis_index("c") * NS + lax.axis_index("s")
    base = wid * EW

    def ex_body(e, carry):
        psum = jnp.zeros((L,), jnp.float32)
        for dsl in range(DIM // L):
            acc = urows_v[e * CTX, pl.ds(dsl * L, L)]
            for cc in range(1, CTX):
                acc = acc + urows_v[e * CTX + cc, pl.ds(dsl * L, L)]
            psum = psum + acc * vrows_v[e, pl.ds(dsl * L, L)]
        psum_v[e, :] = psum
        return carry

    def chunk_body(k, carry):
        cb = base + k * CK
        pltpu.sync_copy(uidx_hbm.at[pl.ds(cb * CTX, CK * CTX)], uidx_v)
        pltpu.sync_copy(vidx_hbm.at[pl.ds(cb, CK)], vidx_v)
        cps = []
        for j in range(0, CK * CTX, 128):
            n = min(128, CK * CTX - j)
            cps.append(pltpu.async_copy(
                u_hbm.at[uidx_v.at[pl.ds(j, n)]],
                urows_v.at[pl.ds(j, n)], sem))
        cps.append(pltpu.async_copy(v_hbm.at[vidx_v], vrows_v, sem))
        for cp in cps:
            cp.wait()
        lax.fori_loop(0, CK, ex_body, 0)
        lane = lax.broadcasted_iota(jnp.int32, (L,), 0)
        tot = jnp.zeros((L,), jnp.float32)
        for j in range(L):
            tot = tot + plsc.load_gather(
                psum_v, [lane, jnp.full((L,), j, jnp.int32)])
        osc_v[...] = tot
        pltpu.sync_copy(osc_v, out_hbm.at[pl.ds(cb, CK)])
        return carry

    lax.fori_loop(0, NCH, chunk_body, 0)


_sc_scores = functools.partial(
    pl.kernel,
    out_type=jax.ShapeDtypeStruct((NTOT,), jnp.float32),
    mesh=plsc.VectorSubcoreMesh(core_axis_name="c", subcore_axis_name="s"),
    scratch_types=[
        pltpu.VMEM((CK * CTX,), jnp.int32),
        pltpu.VMEM((CK,), jnp.int32),
        pltpu.VMEM((CK * CTX, DIM), jnp.float32),
        pltpu.VMEM((CK, DIM), jnp.float32),
        pltpu.VMEM((CK, L), jnp.float32),
        pltpu.VMEM((L,), jnp.float32),
        pltpu.SemaphoreType.DMA,
    ],
    compiler_params=pltpu.CompilerParams(needs_layout_passes=False),
)(_sc_scores_body)


def _tc_loss_body(s_ref, o_ref):
    x = s_ref[...]
    row = lax.broadcasted_iota(jnp.int32, x.shape, 0)
    t = jnp.where(row < B // 128, -x, x)
    sp = jnp.maximum(t, 0.0) + jnp.log1p(jnp.exp(-jnp.abs(t)))
    o_ref[...] = jnp.sum(sp).reshape(1, 1)


def kernel(pos_u, pos_v, neg_u, neg_v, u_emb, v_emb):
    uidx = jnp.concatenate([pos_u.reshape(-1), neg_u.reshape(-1)])
    vidx = jnp.concatenate([pos_v, neg_v])
    scores = _sc_scores(uidx, vidx, u_emb, v_emb)
    loss = pl.pallas_call(
        _tc_loss_body,
        out_shape=jax.ShapeDtypeStruct((1, 1), jnp.float32),
    )(scores.reshape(NTOT // DIM, DIM))
    return loss[0, 0]


# speedup vs baseline: 14.9404x; 1.8400x over previous
import functools

import jax
import jax.numpy as jnp
from jax import lax
from jax.experimental import pallas as pl
from jax.experimental.pallas import tpu as pltpu
from jax.experimental.pallas import tpu_sc as plsc

VOCAB = 100000
DIM = 128
B = 16384
BN = 81920
CTX = 20
NTOT = B + BN

NC = 2
NS = 16
NW = NC * NS
EW = NTOT // NW
CK = 16
NCH = EW // CK
L = 16
UI = CK * CTX


def _sc_scores_body(uidx_hbm, vidx_hbm, u_hbm, v_hbm, out_hbm,
                    ibufA, ibufB, vibufA, vibufB,
                    urows0, urows1, vrows0, vrows1,
                    psum_v, scores_v, sem0, sem1):
    wid = lax.axis_index("c") * NS + lax.axis_index("s")
    base = wid * EW

    def sync_idx(c0, ib, vib):
        ex0 = base + c0 * CK
        pltpu.sync_copy(uidx_hbm.at[pl.ds(ex0 * CTX, 2 * UI)], ib)
        pltpu.sync_copy(vidx_hbm.at[pl.ds(ex0, 2 * CK)], vib)

    def fire(ib, vib, half, ur, vr, sem):
        off = half * UI
        for j in range(0, UI, 128):
            n = min(128, UI - j)
            pltpu.async_copy(u_hbm.at[ib.at[pl.ds(off + j, n)]],
                             ur.at[pl.ds(j, n)], sem)
        pltpu.async_copy(v_hbm.at[vib.at[pl.ds(half * CK, CK)]], vr, sem)

    def drain(ur, vr, sem):
        for j in range(0, UI, 128):
            n = min(128, UI - j)
            pltpu.make_async_copy(u_hbm.at[pl.ds(0, n)],
                                  ur.at[pl.ds(j, n)], sem).wait()
        pltpu.make_async_copy(v_hbm.at[pl.ds(0, CK)], vr, sem).wait()

    def compute(c, ur, vr):
        def ex_body(e, carry):
            psum = jnp.zeros((L,), jnp.float32)
            for dsl in range(DIM // L):
                terms = [ur[e * CTX + cc, pl.ds(dsl * L, L)]
                         for cc in range(CTX)]
                while len(terms) > 1:
                    terms = ([terms[k] + terms[k + 1]
                              for k in range(0, len(terms) - 1, 2)]
                             + ([terms[-1]] if len(terms) % 2 else []))
                psum = psum + terms[0] * vr[e, pl.ds(dsl * L, L)]
            psum_v[e, :] = psum
            return carry

        lax.fori_loop(0, CK, ex_body, 0)
        lane = lax.broadcasted_iota(jnp.int32, (L,), 0)
        tot = jnp.zeros((L,), jnp.float32)
        for j in range(L):
            tot = tot + plsc.load_gather(
                psum_v, [lane, jnp.full((L,), j, jnp.int32)])
        scores_v[pl.ds(c * CK, CK)] = tot

    sync_idx(0, ibufA, vibufA)
    fire(ibufA, vibufA, 0, urows0, vrows0, sem0)

    def body(i, carry):
        c = 4 * i
        fire(ibufA, vibufA, 1, urows1, vrows1, sem1)
        drain(urows0, vrows0, sem0)
        compute(c, urows0, vrows0)
        sync_idx(c + 2, ibufB, vibufB)
        fire(ibufB, vibufB, 0, urows0, vrows0, sem0)
        drain(urows1, vrows1, sem1)
        compute(c + 1, urows1, vrows1)
        fire(ibufB, vibufB, 1, urows1, vrows1, sem1)
        drain(urows0, vrows0, sem0)
        compute(c + 2, urows0, vrows0)
        c4 = jnp.minimum(c + 4, NCH - 2)
        sync_idx(c4, ibufA, vibufA)
        fire(ibufA, vibufA, 0, urows0, vrows0, sem0)
        drain(urows1, vrows1, sem1)
        compute(c + 3, urows1, vrows1)
        return carry

    lax.fori_loop(0, NCH // 4, body, 0)
    drain(urows0, vrows0, sem0)
    pltpu.sync_copy(scores_v, out_hbm.at[pl.ds(base, EW)])


_sc_scores = functools.partial(
    pl.kernel,
    out_type=jax.ShapeDtypeStruct((NTOT,), jnp.float32),
    mesh=plsc.VectorSubcoreMesh(core_axis_name="c", subcore_axis_name="s"),
    scratch_types=[
        pltpu.VMEM((2 * UI,), jnp.int32),
        pltpu.VMEM((2 * UI,), jnp.int32),
        pltpu.VMEM((2 * CK,), jnp.int32),
        pltpu.VMEM((2 * CK,), jnp.int32),
        pltpu.VMEM((UI, DIM), jnp.float32),
        pltpu.VMEM((UI, DIM), jnp.float32),
        pltpu.VMEM((CK, DIM), jnp.float32),
        pltpu.VMEM((CK, DIM), jnp.float32),
        pltpu.VMEM((CK, L), jnp.float32),
        pltpu.VMEM((EW,), jnp.float32),
        pltpu.SemaphoreType.DMA,
        pltpu.SemaphoreType.DMA,
    ],
    compiler_params=pltpu.CompilerParams(needs_layout_passes=False),
)(_sc_scores_body)


def _tc_loss_body(s_ref, o_ref):
    x = s_ref[...]
    row = lax.broadcasted_iota(jnp.int32, x.shape, 0)
    t = jnp.where(row < B // 128, -x, x)
    sp = jnp.maximum(t, 0.0) + jnp.log1p(jnp.exp(-jnp.abs(t)))
    o_ref[...] = jnp.sum(sp).reshape(1, 1)


def kernel(pos_u, pos_v, neg_u, neg_v, u_emb, v_emb):
    uidx = jnp.concatenate([pos_u.reshape(-1), neg_u.reshape(-1)])
    vidx = jnp.concatenate([pos_v, neg_v])
    scores = _sc_scores(uidx, vidx, u_emb, v_emb)
    loss = pl.pallas_call(
        _tc_loss_body,
        out_shape=jax.ShapeDtypeStruct((1, 1), jnp.float32),
    )(scores.reshape(NTOT // DIM, DIM))
    return loss[0, 0]
